# Initial kernel scaffold; baseline (speedup 1.0000x reference)
import functools

import jax
import jax.numpy as jnp
from jax import lax
from jax.experimental import pallas as pl
from jax.experimental.pallas import tpu as pltpu
from jax.experimental.pallas import tpu_sc as plsc

_CHUNK = 128
_NW = 32


def _emb_call(x2d, weight):
    n_rows, chunk = x2d.shape
    _, d = weight.shape
    n = n_rows * chunk
    rows_per_w = n_rows // _NW

    mesh = plsc.VectorSubcoreMesh(core_axis_name="c", subcore_axis_name="s")

    @functools.partial(
        pl.kernel,
        mesh=mesh,
        out_type=jax.ShapeDtypeStruct((n, d), jnp.float32),
        scratch_types=[
            pltpu.VMEM((rows_per_w, chunk), jnp.int32),
            pltpu.VMEM((chunk, d), jnp.float32),
            pltpu.SemaphoreType.DMA,
        ],
    )
    def emb(x_hbm, w_hbm, out_hbm, idx_v, rows_v, gsem):
        wid = lax.axis_index("s") * 2 + lax.axis_index("c")
        row0 = wid * rows_per_w
        pltpu.sync_copy(x_hbm.at[pl.ds(row0, rows_per_w)], idx_v)

        def body(j, carry):
            pltpu.async_copy(w_hbm.at[idx_v.at[j]], rows_v, gsem).wait()
            pltpu.sync_copy(rows_v, out_hbm.at[pl.ds((row0 + j) * chunk, chunk)])
            return carry

        lax.fori_loop(0, rows_per_w, body, 0)

    return emb(x2d, weight)


def kernel(x, weight):
    b, s = x.shape
    d = weight.shape[1]
    x2d = x.reshape(-1).astype(jnp.int32).reshape((b * s) // _CHUNK, _CHUNK)
    out = _emb_call(x2d, weight)
    return out.reshape(b, s, d)


# speedup vs baseline: 4.0897x; 4.0897x over previous
import functools

import jax
import jax.numpy as jnp
from jax import lax
from jax.experimental import pallas as pl
from jax.experimental.pallas import tpu as pltpu
from jax.experimental.pallas import tpu_sc as plsc

_CHUNK = 128
_NW = 32


def _emb_call(x3d, weight):
    nw, rows_per_w, chunk = x3d.shape
    _, d = weight.shape
    n = nw * rows_per_w * chunk

    mesh = plsc.VectorSubcoreMesh(core_axis_name="c", subcore_axis_name="s")

    @functools.partial(
        pl.kernel,
        mesh=mesh,
        out_type=jax.ShapeDtypeStruct((n, d), jnp.float32),
        compiler_params=pltpu.CompilerParams(use_tc_tiling_on_sc=False),
        scratch_types=[
            pltpu.VMEM((rows_per_w, chunk), jnp.int32),
            pltpu.VMEM((chunk, d), jnp.float32),
            pltpu.SemaphoreType.DMA,
        ],
    )
    def emb(x_hbm, w_hbm, out_hbm, idx_v, rows_v, gsem):
        wid = lax.axis_index("s") * 2 + lax.axis_index("c")
        row0 = wid * rows_per_w
        pltpu.sync_copy(x_hbm.at[wid], idx_v)

        def body(j, carry):
            pltpu.async_copy(w_hbm.at[idx_v.at[j]], rows_v, gsem).wait()
            pltpu.sync_copy(rows_v, out_hbm.at[pl.ds((row0 + j) * chunk, chunk)])
            return carry

        lax.fori_loop(0, rows_per_w, body, 0)

    return emb(x3d, weight)


def kernel(x, weight):
    b, s = x.shape
    d = weight.shape[1]
    n = b * s
    x3d = x.reshape(-1).astype(jnp.int32).reshape(_NW, n // (_NW * _CHUNK), _CHUNK)
    out = _emb_call(x3d, weight)
    return out.reshape(b, s, d)


# speedup vs baseline: 4.6568x; 1.1387x over previous
import functools

import jax
import jax.numpy as jnp
from jax import lax
from jax.experimental import pallas as pl
from jax.experimental.pallas import tpu as pltpu
from jax.experimental.pallas import tpu_sc as plsc

_CHUNK = 128
_GROUP = 5
_NBUF = 2
_NW = 32


def _emb_call(x3d, weight):
    nw, n_chunks, chunk = x3d.shape
    _, d = weight.shape
    n = nw * n_chunks * chunk
    groups = n_chunks // _GROUP
    grows = _GROUP * chunk

    mesh = plsc.VectorSubcoreMesh(core_axis_name="c", subcore_axis_name="s")

    @functools.partial(
        pl.kernel,
        mesh=mesh,
        out_type=jax.ShapeDtypeStruct((n, d), jnp.float32),
        compiler_params=pltpu.CompilerParams(use_tc_tiling_on_sc=False),
        scratch_types=[
            pltpu.VMEM((n_chunks, chunk), jnp.int32),
            pltpu.VMEM((_NBUF, grows, d), jnp.float32),
            pltpu.SemaphoreType.DMA((_NBUF,)),
            pltpu.SemaphoreType.DMA((_NBUF,)),
        ],
    )
    def emb(x_hbm, w_hbm, out_hbm, idx_v, rows_v, gsem, ssem):
        wid = lax.axis_index("s") * 2 + lax.axis_index("c")
        base = wid * n_chunks * chunk
        pltpu.sync_copy(x_hbm.at[wid], idx_v)

        def start_gathers(g, p):
            for q in range(_GROUP):
                pltpu.async_copy(
                    w_hbm.at[idx_v.at[g * _GROUP + q]],
                    rows_v.at[p].at[pl.ds(q * chunk, chunk)],
                    gsem.at[p],
                )

        def wait_gathers(p):
            pltpu.make_async_copy(
                w_hbm.at[pl.ds(0, grows)], rows_v.at[p], gsem.at[p]
            ).wait()

        def start_store(g, p):
            pltpu.async_copy(
                rows_v.at[p], out_hbm.at[pl.ds(base + g * grows, grows)], ssem.at[p]
            )

        def wait_store(g, p):
            pltpu.make_async_copy(
                rows_v.at[p], out_hbm.at[pl.ds(base + g * grows, grows)], ssem.at[p]
            ).wait()

        for p in range(_NBUF):
            start_gathers(p, p)

        def body(pp, carry):
            for p in range(_NBUF):
                g = pp * _NBUF + p
                wait_gathers(p)
                start_store(g, p)
                wait_store(g, p)
                start_gathers(g + _NBUF, p)
            return carry

        lax.fori_loop(0, groups // _NBUF - 1, body, 0)

        for p in range(_NBUF):
            g = groups - _NBUF + p
            wait_gathers(p)
            start_store(g, p)
        for p in range(_NBUF):
            wait_store(groups - _NBUF + p, p)

    return emb(x3d, weight)


def kernel(x, weight):
    b, s = x.shape
    d = weight.shape[1]
    n = b * s
    x3d = x.reshape(-1).astype(jnp.int32).reshape(_NW, n // (_NW * _CHUNK), _CHUNK)
    out = _emb_call(x3d, weight)
    return out.reshape(b, s, d)


# speedup vs baseline: 4.6694x; 1.0027x over previous
import functools

import jax
import jax.numpy as jnp
from jax import lax
from jax.experimental import pallas as pl
from jax.experimental.pallas import tpu as pltpu
from jax.experimental.pallas import tpu_sc as plsc

_GROUP = 16
_NBUF = 2
_NW = 32


def _emb_call(x, weight):
    b, s = x.shape
    _, d = weight.shape
    rows_per_tile = b // _NW
    groups = rows_per_tile // _GROUP

    mesh = plsc.VectorSubcoreMesh(core_axis_name="c", subcore_axis_name="s")

    @functools.partial(
        pl.kernel,
        mesh=mesh,
        out_type=jax.ShapeDtypeStruct((b, s, d), jnp.float32),
        compiler_params=pltpu.CompilerParams(use_tc_tiling_on_sc=False),
        scratch_types=[
            pltpu.VMEM((rows_per_tile, s), jnp.int32),
            pltpu.VMEM((_NBUF, _GROUP, s, d), jnp.float32),
            pltpu.SemaphoreType.DMA((_NBUF,)),
            pltpu.SemaphoreType.DMA((_NBUF,)),
        ],
    )
    def emb(x_hbm, w_hbm, out_hbm, idx_v, rows_v, gsem, ssem):
        wid = lax.axis_index("s") * 2 + lax.axis_index("c")
        row0 = wid * rows_per_tile
        pltpu.sync_copy(x_hbm.at[pl.ds(row0, rows_per_tile)], idx_v)

        def start_gathers(g, p):
            for q in range(_GROUP):
                pltpu.async_copy(
                    w_hbm.at[idx_v.at[g * _GROUP + q]],
                    rows_v.at[p].at[q],
                    gsem.at[p],
                )

        def out_slice(g):
            return out_hbm.at[pl.ds(row0 + g * _GROUP, _GROUP)]

        def wait_gathers(g, p):
            pltpu.make_async_copy(out_slice(g), rows_v.at[p], gsem.at[p]).wait()

        def start_store(g, p):
            pltpu.async_copy(rows_v.at[p], out_slice(g), ssem.at[p])

        def wait_store(g, p):
            pltpu.make_async_copy(rows_v.at[p], out_slice(g), ssem.at[p]).wait()

        for p in range(_NBUF):
            start_gathers(p, p)

        def body(pp, carry):
            for p in range(_NBUF):
                g = pp * _NBUF + p
                wait_gathers(g, p)
                start_store(g, p)
                wait_store(g, p)
                start_gathers(g + _NBUF, p)
            return carry

        lax.fori_loop(0, groups // _NBUF - 1, body, 0)

        for p in range(_NBUF):
            g = groups - _NBUF + p
            wait_gathers(g, p)
            start_store(g, p)
        for p in range(_NBUF):
            wait_store(groups - _NBUF + p, p)

    return emb(x, weight)


def kernel(x, weight):
    return _emb_call(x.astype(jnp.int32), weight)
